# baseline (speedup 1.0000x reference)
import functools
import math

import jax
import jax.numpy as jnp
from jax import lax
from jax.experimental import pallas as pl
from jax.experimental.pallas import tpu as pltpu
from jax.experimental.pallas import tpu_sc as plsc

VOCAB = 1000000
D = 64
SCALE = math.sqrt(D)
LANES = 16

NC = 2
NS = 16
NW = NC * NS

B = 4096 * 200
B_PER_W = B // NW
CHUNK = 512
STEPS = B_PER_W // CHUNK


def _body(tok_hbm, table_hbm, out_hbm, idx_v, rows_v, sem_g, sem_o):
    wid = lax.axis_index("s") * NC + lax.axis_index("c")
    base = wid * B_PER_W

    def run_chunk(g, _):
        row0 = base + g * CHUNK
        pltpu.sync_copy(tok_hbm.at[pl.ds(row0, CHUNK)], idx_v)
        pltpu.async_copy(table_hbm.at[idx_v], rows_v, sem_g).wait()

        def scale_row(i, _):
            for j in range(D // LANES):
                sl = pl.ds(j * LANES, LANES)
                rows_v[i, sl] = rows_v[i, sl] * SCALE
            return 0

        lax.fori_loop(0, CHUNK, scale_row, 0)
        pltpu.sync_copy(rows_v, out_hbm.at[pl.ds(row0, CHUNK)])
        return 0

    lax.fori_loop(0, STEPS, run_chunk, 0)


def kernel(tokens, embedding):
    tok_flat = tokens.reshape(-1).astype(jnp.int32)
    mesh = plsc.VectorSubcoreMesh(core_axis_name="c", subcore_axis_name="s")
    out = pl.kernel(
        _body,
        out_type=jax.ShapeDtypeStruct((B, D), jnp.float32),
        mesh=mesh,
        scratch_types=[
            pltpu.VMEM((CHUNK,), jnp.int32),
            pltpu.VMEM((CHUNK, D), jnp.float32),
            pltpu.SemaphoreType.DMA,
            pltpu.SemaphoreType.DMA,
        ],
        compiler_params=pltpu.CompilerParams(use_tc_tiling_on_sc=False),
    )(tok_flat, embedding)
    return out.reshape(tokens.shape + (D,))


# speedup vs baseline: 1.1375x; 1.1375x over previous
import math

import jax
import jax.numpy as jnp
from jax import lax
from jax.experimental import pallas as pl
from jax.experimental.pallas import tpu as pltpu
from jax.experimental.pallas import tpu_sc as plsc

VOCAB = 1000000
D = 64
SCALE = math.sqrt(D)
LANES = 16

NC = 2
NS = 16
NW = NC * NS

B = 4096 * 200
B_PER_W = B // NW
CHUNK = 400
STEPS = B_PER_W // CHUNK
NBUF = 4
LOOKAHEAD = 2


def _body(tok_hbm, table_hbm, out_hbm, idx_v, rows_v, sem_g, sem_o):
    wid = lax.axis_index("s") * NC + lax.axis_index("c")
    base = wid * B_PER_W

    pltpu.sync_copy(tok_hbm.at[pl.ds(base, B_PER_W)], idx_v)

    def gather_start(g, b):
        idx = idx_v.at[pl.ds(g * CHUNK, CHUNK)]
        pltpu.make_async_copy(table_hbm.at[idx], rows_v.at[b], sem_g.at[b]).start()

    def gather_wait(b):
        pltpu.make_async_copy(
            table_hbm.at[idx_v.at[pl.ds(0, CHUNK)]], rows_v.at[b], sem_g.at[b]
        ).wait()

    def out_start(g, b):
        pltpu.make_async_copy(
            rows_v.at[b], out_hbm.at[pl.ds(base + g * CHUNK, CHUNK)], sem_o.at[b]
        ).start()

    def out_wait(b):
        pltpu.make_async_copy(
            rows_v.at[b], out_hbm.at[pl.ds(base, CHUNK)], sem_o.at[b]
        ).wait()

    for g in range(LOOKAHEAD):
        gather_start(g, g % NBUF)

    def outer(go, _):
        for k in range(NBUF):
            g = go * NBUF + k
            gp = g + LOOKAHEAD
            bp = (k + LOOKAHEAD) % NBUF

            @pl.when(jnp.logical_and(gp < STEPS, gp >= NBUF))
            def _():
                out_wait(bp)

            @pl.when(gp < STEPS)
            def _():
                gather_start(gp, bp)

            gather_wait(k)

            rows = rows_v.at[k]

            @plsc.parallel_loop(0, CHUNK, step=1, unroll=8)
            def _(i):
                for j in range(D // LANES):
                    sl = pl.ds(j * LANES, LANES)
                    rows[i, sl] = rows[i, sl] * SCALE

            out_start(g, k)
        return 0

    lax.fori_loop(0, STEPS // NBUF, outer, 0)

    for b in range(NBUF):
        out_wait(b)


def kernel(tokens, embedding):
    tok_flat = tokens.reshape(-1).astype(jnp.int32)
    mesh = plsc.VectorSubcoreMesh(core_axis_name="c", subcore_axis_name="s")
    out = pl.kernel(
        _body,
        out_type=jax.ShapeDtypeStruct((B, D), jnp.float32),
        mesh=mesh,
        scratch_types=[
            pltpu.VMEM((B_PER_W,), jnp.int32),
            pltpu.VMEM((NBUF, CHUNK, D), jnp.float32),
            pltpu.SemaphoreType.DMA((NBUF,)),
            pltpu.SemaphoreType.DMA((NBUF,)),
        ],
        compiler_params=pltpu.CompilerParams(use_tc_tiling_on_sc=False),
    )(tok_flat, embedding)
    return out.reshape(tokens.shape + (D,))
